# baseline (speedup 1.0000x reference)
import jax
import jax.numpy as jnp
from jax import lax
from jax.experimental import pallas as pl
from jax.experimental.pallas import tpu as pltpu
from jax.experimental.pallas import tpu_sc as plsc

_NUM_CORES = 2
_NUM_SUBCORES = 16
_NBUF = 4


def kernel(x, table):
    batch, seq = x.shape
    vocab, emb = table.shape
    nw = _NUM_CORES * _NUM_SUBCORES
    bpw = batch // nw
    nch = bpw

    seq_pad = 64
    idx = jnp.pad(x.astype(jnp.int32), ((0, 0), (0, seq_pad - seq))).reshape(-1)
    mesh = plsc.VectorSubcoreMesh(core_axis_name="c", subcore_axis_name="s")

    seq_st = 56

    @pl.kernel(
        out_type=jax.ShapeDtypeStruct((batch, seq_st, emb), jnp.float32),
        mesh=mesh,
        scratch_types=[
            pltpu.VMEM((bpw * seq_pad,), jnp.int32),
            pltpu.VMEM((_NBUF, seq_st, emb), jnp.float32),
            pltpu.SemaphoreType.DMA((_NBUF,)),
            pltpu.SemaphoreType.DMA((_NBUF,)),
        ],
    )
    def k(table_hbm, i_hbm, o_hbm, idx_v, buf, gsem, osem):
        wid = lax.axis_index("s") * _NUM_CORES + lax.axis_index("c")
        base = wid * bpw
        pltpu.sync_copy(i_hbm.at[pl.ds(base * seq_pad, bpw * seq_pad)], idx_v)

        def g_copy(g, b):
            return pltpu.make_async_copy(
                table_hbm.at[idx_v.at[pl.ds(g * seq_pad, seq_st)]],
                buf.at[b],
                gsem.at[b])

        def o_copy(g, b):
            return pltpu.make_async_copy(
                buf.at[b], o_hbm.at[base + g], osem.at[b])

        for b in range(_NBUF):
            g_copy(b, b).start()
        for g in (0, 1, 2, 3):
            b = g % _NBUF
            g_copy(g, b).wait()
            o_copy(g, b).start()
            if g >= 1:
                o_copy(g - 1, (g - 1) % _NBUF).wait()
                g_copy(g + 3, (g - 1) % _NBUF).start()

        @pl.loop(1, nch // _NBUF - 1)
        def _(c):
            for b in range(_NBUF):
                g = c * _NBUF + b
                g_copy(g, b).wait()
                o_copy(g, b).start()
                o_copy(g - 1, (g - 1) % _NBUF).wait()
                g_copy(g + 3, (g + 3) % _NBUF).start()

        for g in range(nch - _NBUF, nch):
            b = g % _NBUF
            g_copy(g, b).wait()
            o_copy(g, b).start()
            o_copy(g - 1, (g - 1) % _NBUF).wait()
            if g == nch - _NBUF:
                g_copy(nch - 1, (nch - 1) % _NBUF).start()
        o_copy(nch - 1, (nch - 1) % _NBUF).wait()

    return k(table, idx)[:, :seq, :]


# speedup vs baseline: 6.7445x; 6.7445x over previous
import jax
import jax.numpy as jnp
from jax import lax
from jax.experimental import pallas as pl
from jax.experimental.pallas import tpu as pltpu
from jax.experimental.pallas import tpu_sc as plsc

_NUM_CORES = 2
_NUM_SUBCORES = 16
_HALF = 64


def kernel(x, table):
    batch, seq = x.shape
    vocab, emb = table.shape
    nw = _NUM_CORES * _NUM_SUBCORES
    bpw = batch // nw
    seq_pad = seq + (-seq) % 8
    per_w = seq_pad * bpw
    nch = seq * (bpw // _HALF)

    xt = jnp.pad(x.astype(jnp.int32).T, ((0, seq_pad - seq), (0, 0)))
    arr = xt.reshape(seq_pad, nw, bpw).transpose(1, 0, 2).reshape(nw, per_w)

    mesh = plsc.VectorSubcoreMesh(core_axis_name="c", subcore_axis_name="s")

    @pl.kernel(
        out_type=jax.ShapeDtypeStruct((seq, batch, emb), jnp.float32),
        mesh=mesh,
        scratch_types=[
            pltpu.VMEM((per_w,), jnp.int32),
            pltpu.VMEM((2, _HALF, emb), jnp.float32),
            pltpu.SemaphoreType.DMA((2,)),
            pltpu.SemaphoreType.DMA((2,)),
        ],
    )
    def k(table_hbm, i_hbm, o_hbm, idx_v, buf, gsem, osem):
        wid = lax.axis_index("s") * _NUM_CORES + lax.axis_index("c")
        base_b = wid * bpw
        pltpu.sync_copy(i_hbm.at[wid], idx_v)

        def g_copy(g, b):
            return pltpu.make_async_copy(
                table_hbm.at[idx_v.at[pl.ds(g * _HALF, _HALF)]],
                buf.at[b],
                gsem.at[b])

        def o_copy(g, b):
            return pltpu.make_async_copy(
                buf.at[b],
                o_hbm.at[g // 2, pl.ds(base_b + (g % 2) * _HALF, _HALF)],
                osem.at[b])

        g_copy(0, 0).start()
        g_copy(0, 0).wait()
        g_copy(1, 1).start()
        o_copy(0, 0).start()

        @pl.loop(0, (nch - 2) // 2)
        def _(c):
            for u in (0, 1):
                g = 2 + 2 * c + u
                o_copy(g - 2, u).wait()
                g_copy(g, u).start()
                g_copy(g - 1, 1 - u).wait()
                o_copy(g - 1, 1 - u).start()

        g_copy(nch - 1, 1).wait()
        o_copy(nch - 1, 1).start()
        o_copy(nch - 2, 0).wait()
        o_copy(nch - 1, 1).wait()

    return k(table, arr).transpose(1, 0, 2)


# speedup vs baseline: 6.7773x; 1.0049x over previous
import jax
import jax.numpy as jnp
from jax import lax
from jax.experimental import pallas as pl
from jax.experimental.pallas import tpu as pltpu
from jax.experimental.pallas import tpu_sc as plsc

_NUM_CORES = 2
_NUM_SUBCORES = 16
_HALF = 32
_NBUF = 4


def kernel(x, table):
    batch, seq = x.shape
    vocab, emb = table.shape
    nw = _NUM_CORES * _NUM_SUBCORES
    bpw = batch // nw
    seq_pad = seq + (-seq) % 8
    per_w = seq_pad * bpw
    nch = seq * (bpw // _HALF)

    xt = jnp.pad(x.astype(jnp.int32).T, ((0, seq_pad - seq), (0, 0)))
    arr = xt.reshape(seq_pad, nw, bpw).transpose(1, 0, 2).reshape(nw, per_w)

    mesh = plsc.VectorSubcoreMesh(core_axis_name="c", subcore_axis_name="s")

    @pl.kernel(
        out_type=jax.ShapeDtypeStruct((seq, batch, emb), jnp.float32),
        mesh=mesh,
        scratch_types=[
            pltpu.VMEM((per_w,), jnp.int32),
            pltpu.VMEM((_NBUF, _HALF, emb), jnp.float32),
            pltpu.SemaphoreType.DMA((_NBUF,)),
            pltpu.SemaphoreType.DMA((_NBUF,)),
        ],
    )
    def k(table_hbm, i_hbm, o_hbm, idx_v, buf, gsem, osem):
        wid = lax.axis_index("s") * _NUM_CORES + lax.axis_index("c")
        base_b = wid * bpw
        pltpu.sync_copy(i_hbm.at[wid], idx_v)

        def g_copy(g, b):
            return pltpu.make_async_copy(
                table_hbm.at[idx_v.at[pl.ds(g * _HALF, _HALF)]],
                buf.at[b],
                gsem.at[b])

        nq = bpw // _HALF

        def o_copy(g, b):
            return pltpu.make_async_copy(
                buf.at[b],
                o_hbm.at[g // nq, pl.ds(base_b + (g % nq) * _HALF, _HALF)],
                osem.at[b])

        for b in range(_NBUF):
            g_copy(b, b).start()
        for g in (0, 1, 2, 3):
            b = g % _NBUF
            g_copy(g, b).wait()
            o_copy(g, b).start()
            if g >= 1:
                o_copy(g - 1, (g - 1) % _NBUF).wait()
                g_copy(g + 3, (g - 1) % _NBUF).start()

        @pl.loop(1, nch // _NBUF - 1)
        def _(c):
            for u in range(_NBUF):
                g = c * _NBUF + u
                g_copy(g, u).wait()
                o_copy(g, u).start()
                o_copy(g - 1, (g - 1) % _NBUF).wait()
                g_copy(g + 3, (g + 3) % _NBUF).start()

        for g in range(nch - _NBUF, nch):
            b = g % _NBUF
            g_copy(g, b).wait()
            o_copy(g, b).start()
            o_copy(g - 1, (g - 1) % _NBUF).wait()
            if g == nch - _NBUF:
                g_copy(nch - 1, (nch - 1) % _NBUF).start()
        o_copy(nch - 1, (nch - 1) % _NBUF).wait()

    return k(table, arr).transpose(1, 0, 2)
